# Initial kernel scaffold; baseline (speedup 1.0000x reference)
import jax
import jax.numpy as jnp
from jax.experimental import pallas as pl


def kernel(x, edge_index, edge_attr, batch, en1_W1, en1_b1, en1_W2, en1_b2, root1, bias1, en2_W1, en2_b1, en2_W2, en2_b2, root2, bias2, en3_W1, en3_b1, en3_W2, en3_b2, root3, bias3):
    raise NotImplementedError("write your pallas kernel here")


# speedup vs baseline: 2.0946x; 2.0946x over previous
import functools

import jax
import jax.numpy as jnp
import numpy as np
from jax import lax
from jax.experimental import pallas as pl
from jax.experimental.pallas import tpu as pltpu
from jax.experimental.pallas import tpu_sc as plsc

N = 10000
E = 160000
IN = 16
ED = 4
H = 16
OUT = 32
G = 256

NW = 32
CH = 128
E_PAD = 163840
PER_W = E_PAD // NW
N_PAD = 10240
TN = 1000
TE = 1024


def _rs_mats(i_ch, o_ch):
    c = np.arange(i_ch * o_ch)
    r = (c[None, :] // o_ch == np.arange(i_ch)[:, None]).astype(np.float32)
    s = (c[:, None] % o_ch == np.arange(o_ch)[None, :]).astype(np.float32)
    return jnp.asarray(r), jnp.asarray(s)


def _sc_gather(table, idx, d):
    mesh = plsc.VectorSubcoreMesh(core_axis_name="c", subcore_axis_name="s")

    @functools.partial(
        pl.kernel, mesh=mesh,
        out_type=jax.ShapeDtypeStruct((E_PAD, d), jnp.float32),
        compiler_params=pltpu.CompilerParams(use_tc_tiling_on_sc=False),
        scratch_types=[
            pltpu.VMEM((CH,), jnp.int32),
            pltpu.VMEM((CH, d), jnp.float32),
            pltpu.SemaphoreType.DMA,
        ],
    )
    def k(table_hbm, idx_hbm, out_hbm, idx_v, rows_v, sem):
        wid = lax.axis_index("s") * 2 + lax.axis_index("c")
        base = wid * PER_W

        def body(j, carry):
            cb = base + j * CH
            pltpu.sync_copy(idx_hbm.at[pl.ds(cb, CH)], idx_v)
            pltpu.async_copy(table_hbm.at[idx_v], rows_v, sem).wait()
            pltpu.sync_copy(rows_v, out_hbm.at[pl.ds(cb, CH)])
            return carry

        lax.fori_loop(0, PER_W // CH, body, 0)

    return k(table, idx)


def _sc_scatter_add(msg, dst, o_ch, zeros_hbm):
    mesh = plsc.VectorSubcoreMesh(core_axis_name="c", subcore_axis_name="s")
    stripe = N_PAD // 16

    @functools.partial(
        pl.kernel, mesh=mesh,
        out_type=jax.ShapeDtypeStruct((2, N_PAD, o_ch), jnp.float32),
        compiler_params=pltpu.CompilerParams(use_tc_tiling_on_sc=False),
        scratch_types=[
            pltpu.VMEM((CH,), jnp.int32),
            pltpu.VMEM((CH, o_ch), jnp.float32),
            pltpu.VMEM_SHARED((N_PAD, o_ch), jnp.float32),
            pltpu.SemaphoreType.DMA,
        ],
    )
    def k(msg_hbm, dst_hbm, z_hbm, out_hbm, idx_v, msg_v, acc_sh, sem):
        cid = lax.axis_index("c")
        sid = lax.axis_index("s")
        wid = sid * 2 + cid
        r0 = sid * stripe
        pltpu.sync_copy(z_hbm.at[pl.ds(r0, stripe)], acc_sh.at[pl.ds(r0, stripe)])
        plsc.subcore_barrier()
        base = wid * PER_W

        def body(j, carry):
            cb = base + j * CH
            pltpu.sync_copy(dst_hbm.at[pl.ds(cb, CH)], idx_v)
            pltpu.sync_copy(msg_hbm.at[pl.ds(cb, CH)], msg_v)
            pltpu.sync_copy(msg_v, acc_sh.at[idx_v], add=True)
            return carry

        lax.fori_loop(0, PER_W // CH, body, 0)
        plsc.subcore_barrier()
        pltpu.sync_copy(acc_sh.at[pl.ds(r0, stripe)],
                        out_hbm.at[cid].at[pl.ds(r0, stripe)])

    return k(msg, dst, zeros_hbm)


def _dense_msgs(attr, xj, w1, b1, w2, b2, r_m, s_m, o_ch):
    io = w2.shape[1]

    def body(attr_ref, xj_ref, w1_ref, b1_ref, w2_ref, b2_ref, r_ref, s_ref,
             out_ref):
        a = attr_ref[...]
        h = jnp.maximum(
            jnp.dot(a, w1_ref[...], preferred_element_type=jnp.float32)
            + b1_ref[...], 0.0)
        w = jnp.dot(h, w2_ref[...], preferred_element_type=jnp.float32) \
            + b2_ref[...]
        xr = jnp.dot(xj_ref[...], r_ref[...],
                     preferred_element_type=jnp.float32)
        out_ref[...] = jnp.dot(w * xr, s_ref[...],
                               preferred_element_type=jnp.float32)

    return pl.pallas_call(
        body,
        grid=(E_PAD // TE,),
        in_specs=[
            pl.BlockSpec((TE, ED), lambda i: (i, 0)),
            pl.BlockSpec((TE, IN), lambda i: (i, 0)),
            pl.BlockSpec((ED, 256), lambda i: (0, 0)),
            pl.BlockSpec((1, 256), lambda i: (0, 0)),
            pl.BlockSpec((256, io), lambda i: (0, 0)),
            pl.BlockSpec((1, io), lambda i: (0, 0)),
            pl.BlockSpec((IN, io), lambda i: (0, 0)),
            pl.BlockSpec((io, o_ch), lambda i: (0, 0)),
        ],
        out_specs=pl.BlockSpec((TE, o_ch), lambda i: (i, 0)),
        out_shape=jax.ShapeDtypeStruct((E_PAD, o_ch), jnp.float32),
    )(attr, xj, w1, b1.reshape(1, -1), w2, b2.reshape(1, -1), r_m, s_m)


def _combine_relu(agg, h_in, root, bias, o_ch):
    def body(agg_ref, h_ref, root_ref, bias_ref, out_ref):
        a = agg_ref[0] + agg_ref[1]
        r = jnp.dot(h_ref[...], root_ref[...],
                    preferred_element_type=jnp.float32)
        out_ref[...] = jnp.maximum(a + r + bias_ref[...], 0.0)

    return pl.pallas_call(
        body,
        grid=(N // TN,),
        in_specs=[
            pl.BlockSpec((2, TN, o_ch), lambda i: (0, i, 0)),
            pl.BlockSpec((TN, h_in.shape[1]), lambda i: (i, 0)),
            pl.BlockSpec(root.shape, lambda i: (0, 0)),
            pl.BlockSpec((1, o_ch), lambda i: (0, 0)),
        ],
        out_specs=pl.BlockSpec((TN, o_ch), lambda i: (i, 0)),
        out_shape=jax.ShapeDtypeStruct((N, o_ch), jnp.float32),
    )(agg, h_in, root, bias.reshape(1, -1))


def _combine_pool(agg, h_in, root, bias, batch3):
    ngrid = N // TN

    def body(agg_ref, h_ref, root_ref, bias_ref, batch_ref, out_ref,
             sums_scr, cnt_scr):
        pid = pl.program_id(0)
        a = agg_ref[0] + agg_ref[1]
        r = jnp.dot(h_ref[...], root_ref[...],
                    preferred_element_type=jnp.float32)
        h3 = a + r + bias_ref[...]
        b = batch_ref[0]
        gid = lax.broadcasted_iota(jnp.int32, (G, TN), 0)
        onehot = (gid == b).astype(jnp.float32)
        psum = jnp.dot(onehot, h3, preferred_element_type=jnp.float32)
        pcnt = jnp.sum(onehot, axis=1, keepdims=True)

        @pl.when(pid == 0)
        def _():
            sums_scr[...] = psum
            cnt_scr[...] = pcnt

        @pl.when(pid != 0)
        def _():
            sums_scr[...] = sums_scr[...] + psum
            cnt_scr[...] = cnt_scr[...] + pcnt

        out_ref[...] = sums_scr[...] / jnp.maximum(cnt_scr[...], 1.0)

    return pl.pallas_call(
        body,
        grid=(ngrid,),
        in_specs=[
            pl.BlockSpec((2, TN, OUT), lambda i: (0, i, 0)),
            pl.BlockSpec((TN, H), lambda i: (i, 0)),
            pl.BlockSpec((H, OUT), lambda i: (0, 0)),
            pl.BlockSpec((1, OUT), lambda i: (0, 0)),
            pl.BlockSpec((1, 1, TN), lambda i: (i, 0, 0)),
        ],
        out_specs=pl.BlockSpec((G, OUT), lambda i: (0, 0)),
        out_shape=jax.ShapeDtypeStruct((G, OUT), jnp.float32),
        scratch_shapes=[
            pltpu.VMEM((G, OUT), jnp.float32),
            pltpu.VMEM((G, 1), jnp.float32),
        ],
    )(agg, h_in, root, bias.reshape(1, -1), batch3)


def kernel(x, edge_index, edge_attr, batch,
           en1_W1, en1_b1, en1_W2, en1_b2, root1, bias1,
           en2_W1, en2_b1, en2_W2, en2_b2, root2, bias2,
           en3_W1, en3_b1, en3_W2, en3_b2, root3, bias3):
    src = jnp.pad(edge_index[0], (0, E_PAD - E))
    dst = jnp.pad(edge_index[1], (0, E_PAD - E), constant_values=N)
    attr = jnp.pad(edge_attr, ((0, E_PAD - E), (0, 0)))
    batch3 = batch.reshape(N // TN, 1, TN)
    z16 = jnp.zeros((N_PAD, H), jnp.float32)
    z32 = jnp.zeros((N_PAD, OUT), jnp.float32)
    r1, s1 = _rs_mats(IN, H)
    r3, s3 = _rs_mats(H, OUT)

    xj = _sc_gather(x, src, IN)
    msg = _dense_msgs(attr, xj, en1_W1, en1_b1, en1_W2, en1_b2, r1, s1, H)
    agg = _sc_scatter_add(msg, dst, H, z16)
    h1 = _combine_relu(agg[:, :N], x, root1, bias1, H)

    xj = _sc_gather(h1, src, H)
    msg = _dense_msgs(attr, xj, en2_W1, en2_b1, en2_W2, en2_b2, r1, s1, H)
    agg = _sc_scatter_add(msg, dst, H, z16)
    h2 = _combine_relu(agg[:, :N], h1, root2, bias2, H)

    xj = _sc_gather(h2, src, H)
    msg = _dense_msgs(attr, xj, en3_W1, en3_b1, en3_W2, en3_b2, r3, s3, OUT)
    agg = _sc_scatter_add(msg, dst, OUT, z32)
    return _combine_pool(agg[:, :N], h2, root3, bias3, batch3)


# speedup vs baseline: 2.4230x; 1.1568x over previous
import functools

import jax
import jax.numpy as jnp
import numpy as np
from jax import lax
from jax.experimental import pallas as pl
from jax.experimental.pallas import tpu as pltpu
from jax.experimental.pallas import tpu_sc as plsc

N = 10000
E = 160000
IN = 16
ED = 4
H = 16
OUT = 32
G = 256

NW = 32
CH = 128
E_PAD = 163840
PER_W = E_PAD // NW
N_PAD = 10240
TN = 1000
TE = 1024


def _rs_mats(i_ch, o_ch):
    c = np.arange(i_ch * o_ch)
    r = (c[None, :] // o_ch == np.arange(i_ch)[:, None]).astype(np.float32)
    s = (c[:, None] % o_ch == np.arange(o_ch)[None, :]).astype(np.float32)
    return jnp.asarray(r), jnp.asarray(s)


def _sc_gather(table, idx2, d):
    mesh = plsc.VectorSubcoreMesh(core_axis_name="c", subcore_axis_name="s")
    nch = PER_W // CH

    @functools.partial(
        pl.kernel, mesh=mesh,
        out_type=jax.ShapeDtypeStruct((E_PAD, d), jnp.float32),
        compiler_params=pltpu.CompilerParams(use_tc_tiling_on_sc=False),
        scratch_types=[
            pltpu.VMEM((nch, CH), jnp.int32),
            pltpu.VMEM((PER_W, d), jnp.float32),
            pltpu.SemaphoreType.DMA,
        ],
    )
    def k(table_hbm, idx_hbm, out_hbm, idx_v, rows_v, sem):
        wid = lax.axis_index("s") * 2 + lax.axis_index("c")
        pltpu.sync_copy(idx_hbm.at[pl.ds(wid * nch, nch)], idx_v)

        def fire(j, carry):
            pltpu.async_copy(table_hbm.at[idx_v.at[j]],
                             rows_v.at[pl.ds(j * CH, CH)], sem)
            return carry

        def drain(j, carry):
            pltpu.make_async_copy(table_hbm.at[idx_v.at[j]],
                                  rows_v.at[pl.ds(j * CH, CH)], sem).wait()
            return carry

        lax.fori_loop(0, nch, fire, 0)
        lax.fori_loop(0, nch, drain, 0)
        pltpu.sync_copy(rows_v, out_hbm.at[pl.ds(wid * PER_W, PER_W)])

    return k(table, idx2)


def _sc_scatter_add(msg, dst, o_ch, zeros_hbm):
    mesh = plsc.VectorSubcoreMesh(core_axis_name="c", subcore_axis_name="s")
    stripe = N_PAD // 16

    npass = 2 if o_ch > 16 else 1
    p_rows = PER_W // npass
    p_ch = p_rows // CH

    @functools.partial(
        pl.kernel, mesh=mesh,
        out_type=jax.ShapeDtypeStruct((2, N_PAD, o_ch), jnp.float32),
        compiler_params=pltpu.CompilerParams(use_tc_tiling_on_sc=False),
        scratch_types=[
            pltpu.VMEM((p_ch, CH), jnp.int32),
            pltpu.VMEM((p_rows, o_ch), jnp.float32),
            pltpu.VMEM_SHARED((N_PAD, o_ch), jnp.float32),
            pltpu.SemaphoreType.DMA,
        ],
    )
    def k(msg_hbm, dst_hbm, z_hbm, out_hbm, idx_v, msg_v, acc_sh, sem):
        cid = lax.axis_index("c")
        sid = lax.axis_index("s")
        wid = sid * 2 + cid
        r0 = sid * stripe
        pltpu.sync_copy(z_hbm.at[pl.ds(r0, stripe)], acc_sh.at[pl.ds(r0, stripe)])
        plsc.subcore_barrier()

        for p in range(npass):
            rbase = wid * PER_W + p * p_rows
            pltpu.sync_copy(dst_hbm.at[pl.ds(rbase // CH, p_ch)], idx_v)
            pltpu.sync_copy(msg_hbm.at[pl.ds(rbase, p_rows)], msg_v)

            def fire(j, carry):
                pltpu.async_copy(msg_v.at[pl.ds(j * CH, CH)],
                                 acc_sh.at[idx_v.at[j]], sem, add=True)
                return carry

            def drain(j, carry):
                pltpu.make_async_copy(msg_v.at[pl.ds(j * CH, CH)],
                                      acc_sh.at[idx_v.at[j]], sem).wait()
                return carry

            lax.fori_loop(0, p_ch, fire, 0)
            lax.fori_loop(0, p_ch, drain, 0)

        plsc.subcore_barrier()
        pltpu.sync_copy(acc_sh.at[pl.ds(r0, stripe)],
                        out_hbm.at[cid].at[pl.ds(r0, stripe)])

    return k(msg, dst, zeros_hbm)


def _dense_msgs(attr, xj, w1, b1, w2, b2, r_m, s_m, o_ch):
    io = w2.shape[1]

    def body(attr_ref, xj_ref, w1_ref, b1_ref, w2_ref, b2_ref, r_ref, s_ref,
             out_ref):
        a = attr_ref[...]
        h = jnp.maximum(
            jnp.dot(a, w1_ref[...], preferred_element_type=jnp.float32)
            + b1_ref[...], 0.0)
        w = jnp.dot(h, w2_ref[...], preferred_element_type=jnp.float32) \
            + b2_ref[...]
        xr = jnp.dot(xj_ref[...], r_ref[...],
                     preferred_element_type=jnp.float32)
        out_ref[...] = jnp.dot(w * xr, s_ref[...],
                               preferred_element_type=jnp.float32)

    return pl.pallas_call(
        body,
        grid=(E_PAD // TE,),
        in_specs=[
            pl.BlockSpec((TE, ED), lambda i: (i, 0)),
            pl.BlockSpec((TE, IN), lambda i: (i, 0)),
            pl.BlockSpec((ED, 256), lambda i: (0, 0)),
            pl.BlockSpec((1, 256), lambda i: (0, 0)),
            pl.BlockSpec((256, io), lambda i: (0, 0)),
            pl.BlockSpec((1, io), lambda i: (0, 0)),
            pl.BlockSpec((IN, io), lambda i: (0, 0)),
            pl.BlockSpec((io, o_ch), lambda i: (0, 0)),
        ],
        out_specs=pl.BlockSpec((TE, o_ch), lambda i: (i, 0)),
        out_shape=jax.ShapeDtypeStruct((E_PAD, o_ch), jnp.float32),
    )(attr, xj, w1, b1.reshape(1, -1), w2, b2.reshape(1, -1), r_m, s_m)


def _combine_relu(agg, h_in, root, bias, o_ch):
    def body(agg_ref, h_ref, root_ref, bias_ref, out_ref):
        a = agg_ref[0] + agg_ref[1]
        r = jnp.dot(h_ref[...], root_ref[...],
                    preferred_element_type=jnp.float32)
        out_ref[...] = jnp.maximum(a + r + bias_ref[...], 0.0)

    return pl.pallas_call(
        body,
        grid=(N // TN,),
        in_specs=[
            pl.BlockSpec((2, TN, o_ch), lambda i: (0, i, 0)),
            pl.BlockSpec((TN, h_in.shape[1]), lambda i: (i, 0)),
            pl.BlockSpec(root.shape, lambda i: (0, 0)),
            pl.BlockSpec((1, o_ch), lambda i: (0, 0)),
        ],
        out_specs=pl.BlockSpec((TN, o_ch), lambda i: (i, 0)),
        out_shape=jax.ShapeDtypeStruct((N, o_ch), jnp.float32),
    )(agg, h_in, root, bias.reshape(1, -1))


def _combine_pool(agg, h_in, root, bias, batch3):
    ngrid = N // TN

    def body(agg_ref, h_ref, root_ref, bias_ref, batch_ref, out_ref,
             sums_scr, cnt_scr):
        pid = pl.program_id(0)
        a = agg_ref[0] + agg_ref[1]
        r = jnp.dot(h_ref[...], root_ref[...],
                    preferred_element_type=jnp.float32)
        h3 = a + r + bias_ref[...]
        b = batch_ref[0]
        gid = lax.broadcasted_iota(jnp.int32, (G, TN), 0)
        onehot = (gid == b).astype(jnp.float32)
        psum = jnp.dot(onehot, h3, preferred_element_type=jnp.float32)
        pcnt = jnp.sum(onehot, axis=1, keepdims=True)

        @pl.when(pid == 0)
        def _():
            sums_scr[...] = psum
            cnt_scr[...] = pcnt

        @pl.when(pid != 0)
        def _():
            sums_scr[...] = sums_scr[...] + psum
            cnt_scr[...] = cnt_scr[...] + pcnt

        out_ref[...] = sums_scr[...] / jnp.maximum(cnt_scr[...], 1.0)

    return pl.pallas_call(
        body,
        grid=(ngrid,),
        in_specs=[
            pl.BlockSpec((2, TN, OUT), lambda i: (0, i, 0)),
            pl.BlockSpec((TN, H), lambda i: (i, 0)),
            pl.BlockSpec((H, OUT), lambda i: (0, 0)),
            pl.BlockSpec((1, OUT), lambda i: (0, 0)),
            pl.BlockSpec((1, 1, TN), lambda i: (i, 0, 0)),
        ],
        out_specs=pl.BlockSpec((G, OUT), lambda i: (0, 0)),
        out_shape=jax.ShapeDtypeStruct((G, OUT), jnp.float32),
        scratch_shapes=[
            pltpu.VMEM((G, OUT), jnp.float32),
            pltpu.VMEM((G, 1), jnp.float32),
        ],
    )(agg, h_in, root, bias.reshape(1, -1), batch3)


def kernel(x, edge_index, edge_attr, batch,
           en1_W1, en1_b1, en1_W2, en1_b2, root1, bias1,
           en2_W1, en2_b1, en2_W2, en2_b2, root2, bias2,
           en3_W1, en3_b1, en3_W2, en3_b2, root3, bias3):
    src = jnp.pad(edge_index[0], (0, E_PAD - E)).reshape(E_PAD // CH, CH)
    dst = jnp.pad(edge_index[1], (0, E_PAD - E),
                  constant_values=N).reshape(E_PAD // CH, CH)
    attr = jnp.pad(edge_attr, ((0, E_PAD - E), (0, 0)))
    batch3 = batch.reshape(N // TN, 1, TN)
    z16 = jnp.zeros((N_PAD, H), jnp.float32)
    z32 = jnp.zeros((N_PAD, OUT), jnp.float32)
    r1, s1 = _rs_mats(IN, H)
    r3, s3 = _rs_mats(H, OUT)

    xj = _sc_gather(x, src, IN)
    msg = _dense_msgs(attr, xj, en1_W1, en1_b1, en1_W2, en1_b2, r1, s1, H)
    agg = _sc_scatter_add(msg, dst, H, z16)
    h1 = _combine_relu(agg[:, :N], x, root1, bias1, H)

    xj = _sc_gather(h1, src, H)
    msg = _dense_msgs(attr, xj, en2_W1, en2_b1, en2_W2, en2_b2, r1, s1, H)
    agg = _sc_scatter_add(msg, dst, H, z16)
    h2 = _combine_relu(agg[:, :N], h1, root2, bias2, H)

    xj = _sc_gather(h2, src, H)
    msg = _dense_msgs(attr, xj, en3_W1, en3_b1, en3_W2, en3_b2, r3, s3, OUT)
    agg = _sc_scatter_add(msg, dst, OUT, z32)
    return _combine_pool(agg[:, :N], h2, root3, bias3, batch3)


# speedup vs baseline: 2.7207x; 1.1229x over previous
import functools

import jax
import jax.numpy as jnp
import numpy as np
from jax import lax
from jax.experimental import pallas as pl
from jax.experimental.pallas import tpu as pltpu
from jax.experimental.pallas import tpu_sc as plsc

N = 10000
E = 160000
IN = 16
ED = 4
H = 16
OUT = 32
G = 256

NW = 32
CH = 128
E_PAD = 163840
PER_W = E_PAD // NW
N_PAD = 10240
TN = 1000
TE = 2048


def _rs_mats(i_ch, o_ch):
    c = np.arange(i_ch * o_ch)
    r = (c[None, :] // o_ch == np.arange(i_ch)[:, None]).astype(np.float32)
    s = (c[:, None] % o_ch == np.arange(o_ch)[None, :]).astype(np.float32)
    return jnp.asarray(r), jnp.asarray(s)


def _sc_gather(table, idx2, d):
    mesh = plsc.VectorSubcoreMesh(core_axis_name="c", subcore_axis_name="s")
    nch = PER_W // CH

    @functools.partial(
        pl.kernel, mesh=mesh,
        out_type=jax.ShapeDtypeStruct((E_PAD, d), jnp.float32),
        compiler_params=pltpu.CompilerParams(use_tc_tiling_on_sc=False),
        scratch_types=[
            pltpu.VMEM((nch, CH), jnp.int32),
            pltpu.VMEM((PER_W, d), jnp.float32),
            pltpu.SemaphoreType.DMA,
        ],
    )
    def k(table_hbm, idx_hbm, out_hbm, idx_v, rows_v, sem):
        wid = lax.axis_index("s") * 2 + lax.axis_index("c")
        pltpu.sync_copy(idx_hbm.at[pl.ds(wid * nch, nch)], idx_v)

        def fire(j, carry):
            pltpu.async_copy(table_hbm.at[idx_v.at[j]],
                             rows_v.at[pl.ds(j * CH, CH)], sem)
            return carry

        def drain(j, carry):
            pltpu.make_async_copy(table_hbm.at[idx_v.at[j]],
                                  rows_v.at[pl.ds(j * CH, CH)], sem).wait()
            return carry

        lax.fori_loop(0, nch, fire, 0)
        lax.fori_loop(0, nch, drain, 0)
        pltpu.sync_copy(rows_v, out_hbm.at[pl.ds(wid * PER_W, PER_W)])

    return k(table, idx2)


def _sc_scatter_add(msg, dst, o_ch, zeros_hbm):
    mesh = plsc.VectorSubcoreMesh(core_axis_name="c", subcore_axis_name="s")
    stripe = N_PAD // 16

    npass = 2 if o_ch > 16 else 1
    p_rows = PER_W // npass
    p_ch = p_rows // CH

    @functools.partial(
        pl.kernel, mesh=mesh,
        out_type=jax.ShapeDtypeStruct((2, N_PAD, o_ch), jnp.float32),
        compiler_params=pltpu.CompilerParams(use_tc_tiling_on_sc=False),
        scratch_types=[
            pltpu.VMEM((p_ch, CH), jnp.int32),
            pltpu.VMEM((p_rows, o_ch), jnp.float32),
            pltpu.VMEM_SHARED((N_PAD, o_ch), jnp.float32),
            pltpu.SemaphoreType.DMA,
        ],
    )
    def k(msg_hbm, dst_hbm, z_hbm, out_hbm, idx_v, msg_v, acc_sh, sem):
        cid = lax.axis_index("c")
        sid = lax.axis_index("s")
        wid = sid * 2 + cid
        r0 = sid * stripe
        pltpu.sync_copy(z_hbm.at[pl.ds(r0, stripe)], acc_sh.at[pl.ds(r0, stripe)])
        plsc.subcore_barrier()

        for p in range(npass):
            rbase = wid * PER_W + p * p_rows
            pltpu.sync_copy(dst_hbm.at[pl.ds(rbase // CH, p_ch)], idx_v)
            pltpu.sync_copy(msg_hbm.at[pl.ds(rbase, p_rows)], msg_v)

            def fire(j, carry):
                pltpu.async_copy(msg_v.at[pl.ds(j * CH, CH)],
                                 acc_sh.at[idx_v.at[j]], sem, add=True)
                return carry

            def drain(j, carry):
                pltpu.make_async_copy(msg_v.at[pl.ds(j * CH, CH)],
                                      acc_sh.at[idx_v.at[j]], sem).wait()
                return carry

            lax.fori_loop(0, p_ch, fire, 0)
            lax.fori_loop(0, p_ch, drain, 0)

        plsc.subcore_barrier()
        pltpu.sync_copy(acc_sh.at[pl.ds(r0, stripe)],
                        out_hbm.at[cid].at[pl.ds(r0, stripe)])

    return k(msg, dst, zeros_hbm)


def _dense_msgs(attr, xj, w1, b1, w2, b2, r_m, s_m, o_ch):
    io = w2.shape[1]

    def body(attr_ref, xj_ref, w1_ref, b1_ref, w2_ref, b2_ref, r_ref, s_ref,
             out_ref):
        a = attr_ref[...]
        h = jnp.maximum(
            jnp.dot(a, w1_ref[...], preferred_element_type=jnp.float32)
            + b1_ref[...], 0.0)
        w = jnp.dot(h.astype(jnp.bfloat16), w2_ref[...].astype(jnp.bfloat16),
                    preferred_element_type=jnp.float32) + b2_ref[...]
        xr = jnp.dot(xj_ref[...], r_ref[...],
                     preferred_element_type=jnp.float32)
        out_ref[...] = jnp.dot(w * xr, s_ref[...],
                               preferred_element_type=jnp.float32)

    return pl.pallas_call(
        body,
        grid=(E_PAD // TE,),
        in_specs=[
            pl.BlockSpec((TE, ED), lambda i: (i, 0)),
            pl.BlockSpec((TE, IN), lambda i: (i, 0)),
            pl.BlockSpec((ED, 256), lambda i: (0, 0)),
            pl.BlockSpec((1, 256), lambda i: (0, 0)),
            pl.BlockSpec((256, io), lambda i: (0, 0)),
            pl.BlockSpec((1, io), lambda i: (0, 0)),
            pl.BlockSpec((IN, io), lambda i: (0, 0)),
            pl.BlockSpec((io, o_ch), lambda i: (0, 0)),
        ],
        out_specs=pl.BlockSpec((TE, o_ch), lambda i: (i, 0)),
        out_shape=jax.ShapeDtypeStruct((E_PAD, o_ch), jnp.float32),
    )(attr, xj, w1, b1.reshape(1, -1), w2, b2.reshape(1, -1), r_m, s_m)


def _combine_relu(agg, h_in, root, bias, o_ch):
    def body(agg_ref, h_ref, root_ref, bias_ref, out_ref):
        a = agg_ref[0] + agg_ref[1]
        r = jnp.dot(h_ref[...], root_ref[...],
                    preferred_element_type=jnp.float32)
        out_ref[...] = jnp.maximum(a + r + bias_ref[...], 0.0)

    return pl.pallas_call(
        body,
        grid=(N // TN,),
        in_specs=[
            pl.BlockSpec((2, TN, o_ch), lambda i: (0, i, 0)),
            pl.BlockSpec((TN, h_in.shape[1]), lambda i: (i, 0)),
            pl.BlockSpec(root.shape, lambda i: (0, 0)),
            pl.BlockSpec((1, o_ch), lambda i: (0, 0)),
        ],
        out_specs=pl.BlockSpec((TN, o_ch), lambda i: (i, 0)),
        out_shape=jax.ShapeDtypeStruct((N, o_ch), jnp.float32),
    )(agg, h_in, root, bias.reshape(1, -1))


def _combine_pool(agg, h_in, root, bias, batch3):
    ngrid = N // TN

    def body(agg_ref, h_ref, root_ref, bias_ref, batch_ref, out_ref,
             sums_scr, cnt_scr):
        pid = pl.program_id(0)
        a = agg_ref[0] + agg_ref[1]
        r = jnp.dot(h_ref[...], root_ref[...],
                    preferred_element_type=jnp.float32)
        h3 = a + r + bias_ref[...]
        b = batch_ref[0]
        gid = lax.broadcasted_iota(jnp.int32, (G, TN), 0)
        onehot = (gid == b).astype(jnp.float32)
        psum = jnp.dot(onehot, h3, preferred_element_type=jnp.float32)
        pcnt = jnp.sum(onehot, axis=1, keepdims=True)

        @pl.when(pid == 0)
        def _():
            sums_scr[...] = psum
            cnt_scr[...] = pcnt

        @pl.when(pid != 0)
        def _():
            sums_scr[...] = sums_scr[...] + psum
            cnt_scr[...] = cnt_scr[...] + pcnt

        out_ref[...] = sums_scr[...] / jnp.maximum(cnt_scr[...], 1.0)

    return pl.pallas_call(
        body,
        grid=(ngrid,),
        in_specs=[
            pl.BlockSpec((2, TN, OUT), lambda i: (0, i, 0)),
            pl.BlockSpec((TN, H), lambda i: (i, 0)),
            pl.BlockSpec((H, OUT), lambda i: (0, 0)),
            pl.BlockSpec((1, OUT), lambda i: (0, 0)),
            pl.BlockSpec((1, 1, TN), lambda i: (i, 0, 0)),
        ],
        out_specs=pl.BlockSpec((G, OUT), lambda i: (0, 0)),
        out_shape=jax.ShapeDtypeStruct((G, OUT), jnp.float32),
        scratch_shapes=[
            pltpu.VMEM((G, OUT), jnp.float32),
            pltpu.VMEM((G, 1), jnp.float32),
        ],
    )(agg, h_in, root, bias.reshape(1, -1), batch3)


def kernel(x, edge_index, edge_attr, batch,
           en1_W1, en1_b1, en1_W2, en1_b2, root1, bias1,
           en2_W1, en2_b1, en2_W2, en2_b2, root2, bias2,
           en3_W1, en3_b1, en3_W2, en3_b2, root3, bias3):
    src = jnp.pad(edge_index[0], (0, E_PAD - E)).reshape(E_PAD // CH, CH)
    dst = jnp.pad(edge_index[1], (0, E_PAD - E),
                  constant_values=N).reshape(E_PAD // CH, CH)
    attr = jnp.pad(edge_attr, ((0, E_PAD - E), (0, 0)))
    batch3 = batch.reshape(N // TN, 1, TN)
    z16 = jnp.zeros((N_PAD, H), jnp.float32)
    z32 = jnp.zeros((N_PAD, OUT), jnp.float32)
    r1, s1 = _rs_mats(IN, H)
    r3, s3 = _rs_mats(H, OUT)

    xj = _sc_gather(x, src, IN)
    msg = _dense_msgs(attr, xj, en1_W1, en1_b1, en1_W2, en1_b2, r1, s1, H)
    agg = _sc_scatter_add(msg, dst, H, z16)
    h1 = _combine_relu(agg[:, :N], x, root1, bias1, H)

    xj = _sc_gather(h1, src, H)
    msg = _dense_msgs(attr, xj, en2_W1, en2_b1, en2_W2, en2_b2, r1, s1, H)
    agg = _sc_scatter_add(msg, dst, H, z16)
    h2 = _combine_relu(agg[:, :N], h1, root2, bias2, H)

    xj = _sc_gather(h2, src, H)
    msg = _dense_msgs(attr, xj, en3_W1, en3_b1, en3_W2, en3_b2, r3, s3, OUT)
    agg = _sc_scatter_add(msg, dst, OUT, z32)
    return _combine_pool(agg[:, :N], h2, root3, bias3, batch3)
